# Initial kernel scaffold; baseline (speedup 1.0000x reference)
import jax
import jax.numpy as jnp
from jax.experimental import pallas as pl


def kernel(x, edge_index, edge_attr, W_M, b_M, W_U, b_U, W_mix, b_mix, gamma_t, beta_t, gamma_o, beta_o):
    raise NotImplementedError("write your pallas kernel here")


# speedup vs baseline: 1.4782x; 1.4782x over previous
import functools

import jax
import jax.numpy as jnp
import numpy as np
from jax.experimental import pallas as pl
from jax.experimental.pallas import tpu as pltpu

DELTA = 2.5
NUM_LAYERS = 4
BN_EPS = 1e-5


def _edge_body(src_ref, dst_ref, a_ref, b_ref,
               s_ref, mx_ref, mn_ref, sq_ref, dg_ref, *, eblk):
    @pl.when(pl.program_id(0) == 0)
    def _init():
        s_ref[...] = jnp.zeros_like(s_ref)
        sq_ref[...] = jnp.zeros_like(sq_ref)
        dg_ref[...] = jnp.zeros_like(dg_ref)
        mx_ref[...] = jnp.full_like(mx_ref, -jnp.inf)
        mn_ref[...] = jnp.full_like(mn_ref, jnp.inf)

    def body(i, carry):
        s = src_ref[0, 0, i]
        d = dst_ref[0, 0, i]
        m = a_ref[pl.ds(s, 1), :] + b_ref[pl.ds(d, 1), :]
        s_ref[pl.ds(d, 1), :] += m
        sq_ref[pl.ds(d, 1), :] += m * m
        mx_ref[pl.ds(d, 1), :] = jnp.maximum(mx_ref[pl.ds(d, 1), :], m)
        mn_ref[pl.ds(d, 1), :] = jnp.minimum(mn_ref[pl.ds(d, 1), :], m)
        dg_ref[pl.ds(d, 1), :] += 1.0
        return carry

    jax.lax.fori_loop(0, eblk, body, 0)


def _edge_aggregate(src3, dst3, A, B):
    n, d = A.shape
    nblk, _, eblk = src3.shape
    out = jax.ShapeDtypeStruct((n, d), jnp.float32)
    full = pl.BlockSpec((n, d), lambda j: (0, 0))
    idx_spec = pl.BlockSpec((1, 1, eblk), lambda j: (j, 0, 0),
                            memory_space=pltpu.SMEM)
    return pl.pallas_call(
        functools.partial(_edge_body, eblk=eblk),
        grid=(nblk,),
        in_specs=[idx_spec, idx_spec, full, full],
        out_specs=[full, full, full, full, full],
        out_shape=[out, out, out, out, out],
    )(src3, dst3, A, B)


def _k1_body(x_ref, s_ref, mx_ref, mn_ref, sq_ref, dg_ref,
             wu_ref, bu_ref, h_ref, ps_ref, pq_ref, *, n_nodes):
    deg = dg_ref[:, 0:1]
    degc = jnp.maximum(deg, 1.0)
    sm = s_ref[...]
    mean = sm / degc
    mx = mx_ref[...]
    mx = jnp.where(jnp.isfinite(mx), mx, 0.0)
    mn = mn_ref[...]
    mn = jnp.where(jnp.isfinite(mn), mn, 0.0)
    mean_sq = sq_ref[...] / degc
    var = jnp.maximum(mean_sq - mean * mean, 0.0)
    std = jnp.sqrt(var + 1e-30)

    logd = jnp.log(deg + 1.0)
    amp = logd / DELTA
    att = jnp.where(logd > 0, DELTA / jnp.maximum(logd, 1e-12), 0.0)

    x = x_ref[...]
    dim = x.shape[1]

    def blockw(k):
        return wu_ref[pl.ds(k * dim, dim), :]

    h = jnp.dot(x, blockw(0), preferred_element_type=jnp.float32)
    aggs = (mean, mx, mn, std, var, sm)
    for a, g in enumerate(aggs):
        h += jnp.dot(g, blockw(1 + a), preferred_element_type=jnp.float32)
        h += jnp.dot(g * amp, blockw(7 + a), preferred_element_type=jnp.float32)
        h += jnp.dot(g * att, blockw(13 + a), preferred_element_type=jnp.float32)
    h = (h + bu_ref[...]) * np.float32(np.sqrt(1.0 / n_nodes))
    h_ref[...] = h

    @pl.when(pl.program_id(0) == 0)
    def _init():
        ps_ref[...] = jnp.zeros_like(ps_ref)
        pq_ref[...] = jnp.zeros_like(pq_ref)

    ps_ref[...] += jnp.sum(h, axis=0, keepdims=True)
    pq_ref[...] += jnp.sum(h * h, axis=0, keepdims=True)


def _k2_body(h_ref, x_ref, mu_ref, mq_ref, gt_ref, bt_ref,
             wm_ref, bm_ref, y_ref, ps_ref, pq_ref, *, n_nodes):
    mu = mu_ref[...] / n_nodes
    var = mq_ref[...] / n_nodes - mu * mu
    h = h_ref[...]
    h = gt_ref[...] * (h - mu) * jax.lax.rsqrt(var + BN_EPS) + bt_ref[...]
    h = jnp.dot(h, wm_ref[...], preferred_element_type=jnp.float32) + bm_ref[...]
    h = jnp.where(h >= 0, h, 0.01 * h)
    y = h + x_ref[...]
    y_ref[...] = y

    @pl.when(pl.program_id(0) == 0)
    def _init():
        ps_ref[...] = jnp.zeros_like(ps_ref)
        pq_ref[...] = jnp.zeros_like(pq_ref)

    ps_ref[...] += jnp.sum(y, axis=0, keepdims=True)
    pq_ref[...] += jnp.sum(y * y, axis=0, keepdims=True)


def _k3_body(y_ref, x_ref, mu_ref, mq_ref, go_ref, bo_ref,
             w1_ref, w2_ref, bmsg_ref, h_ref, a_ref, b_ref, *, n_nodes):
    mu = mu_ref[...] / n_nodes
    var = mq_ref[...] / n_nodes - mu * mu
    y = y_ref[...]
    z = go_ref[...] * (y - mu) * jax.lax.rsqrt(var + BN_EPS) + bo_ref[...]
    h = jnp.maximum(z, 0.0) + x_ref[...]
    h_ref[...] = h
    a_ref[...] = jnp.dot(h, w1_ref[...], preferred_element_type=jnp.float32)
    b_ref[...] = (jnp.dot(h, w2_ref[...], preferred_element_type=jnp.float32)
                  + bmsg_ref[...])


def _ab_body(x_ref, w1_ref, w2_ref, bmsg_ref, a_ref, b_ref):
    x = x_ref[...]
    a_ref[...] = jnp.dot(x, w1_ref[...], preferred_element_type=jnp.float32)
    b_ref[...] = (jnp.dot(x, w2_ref[...], preferred_element_type=jnp.float32)
                  + bmsg_ref[...])


def _row_blocked(body, n, d, rblk, extra_in_specs, extra_outs):
    rb = pl.BlockSpec((rblk, d), lambda i: (i, 0))
    return rb


def kernel(x, edge_index, edge_attr, W_M, b_M, W_U, b_U, W_mix, b_mix,
           gamma_t, beta_t, gamma_o, beta_o):
    n, d = x.shape
    e = edge_index.shape[1]

    w1 = W_M[:, :d].T
    w2 = W_M[:, d:].T
    bm = b_M[None, :]
    wu = W_U.T
    bu = b_U[None, :]
    wmix = W_mix.T
    bmix = b_mix[None, :]
    gt = gamma_t[None, :]
    bt = beta_t[None, :]
    go = gamma_o[None, :]
    bo = beta_o[None, :]

    eblk = 4000 if e % 4000 == 0 else e
    nblk = e // eblk
    src3 = edge_index[0].reshape(nblk, 1, eblk)
    dst3 = edge_index[1].reshape(nblk, 1, eblk)

    rblk = 2000 if n % 2000 == 0 else n
    rgrid = n // rblk
    rb = pl.BlockSpec((rblk, d), lambda i: (i, 0))
    one = pl.BlockSpec((1, d), lambda i: (0, 0))
    wu_spec = pl.BlockSpec(wu.shape, lambda i: (0, 0))
    sq_spec = pl.BlockSpec((d, d), lambda i: (0, 0))
    f32 = jnp.float32

    k1 = pl.pallas_call(
        functools.partial(_k1_body, n_nodes=n),
        grid=(rgrid,),
        in_specs=[rb, rb, rb, rb, rb, rb, wu_spec, one],
        out_specs=[rb, one, one],
        out_shape=[jax.ShapeDtypeStruct((n, d), f32),
                   jax.ShapeDtypeStruct((1, d), f32),
                   jax.ShapeDtypeStruct((1, d), f32)],
    )

    k2 = pl.pallas_call(
        functools.partial(_k2_body, n_nodes=n),
        grid=(rgrid,),
        in_specs=[rb, rb, one, one, one, one, sq_spec, one],
        out_specs=[rb, one, one],
        out_shape=[jax.ShapeDtypeStruct((n, d), f32),
                   jax.ShapeDtypeStruct((1, d), f32),
                   jax.ShapeDtypeStruct((1, d), f32)],
    )

    k3 = pl.pallas_call(
        functools.partial(_k3_body, n_nodes=n),
        grid=(rgrid,),
        in_specs=[rb, rb, one, one, one, one, sq_spec, sq_spec, one],
        out_specs=[rb, rb, rb],
        out_shape=[jax.ShapeDtypeStruct((n, d), f32),
                   jax.ShapeDtypeStruct((n, d), f32),
                   jax.ShapeDtypeStruct((n, d), f32)],
    )

    ab0 = pl.pallas_call(
        _ab_body,
        grid=(rgrid,),
        in_specs=[rb, sq_spec, sq_spec, one],
        out_specs=[rb, rb],
        out_shape=[jax.ShapeDtypeStruct((n, d), f32),
                   jax.ShapeDtypeStruct((n, d), f32)],
    )

    h = x
    A, B = ab0(x, w1, w2, bm)
    for _ in range(NUM_LAYERS):
        s, mx, mn, sq, dg = _edge_aggregate(src3, dst3, A, B)
        hpre, ps, pq = k1(h, s, mx, mn, sq, dg, wu, bu)
        y, ys, yq = k2(hpre, h, ps, pq, gt, bt, wmix, bmix)
        h, A, B = k3(y, h, ys, yq, go, bo, w1, w2, bm)
    return h


# speedup vs baseline: 1.5380x; 1.0404x over previous
import functools

import jax
import jax.numpy as jnp
import numpy as np
from jax.experimental import pallas as pl
from jax.experimental.pallas import tpu as pltpu

DELTA = 2.5
NUM_LAYERS = 4
BN_EPS = 1e-5


def _edge_body(*refs, eblk, with_deg):
    if with_deg:
        (src_ref, dst_ref, a_ref, b_ref,
         s_ref, mx_ref, mn_ref, sq_ref, dg_ref) = refs
    else:
        src_ref, dst_ref, a_ref, b_ref, s_ref, mx_ref, mn_ref, sq_ref = refs
        dg_ref = None

    @pl.when(pl.program_id(1) == 0)
    def _init():
        s_ref[...] = jnp.zeros_like(s_ref)
        sq_ref[...] = jnp.zeros_like(sq_ref)
        mx_ref[...] = jnp.full_like(mx_ref, -jnp.inf)
        mn_ref[...] = jnp.full_like(mn_ref, jnp.inf)
        if dg_ref is not None:
            dg_ref[...] = jnp.zeros_like(dg_ref)

    def body(i, carry):
        s = src_ref[0, 0, 0, i]
        d = dst_ref[0, 0, 0, i]
        m = a_ref[pl.ds(s, 1), :] + b_ref[pl.ds(d, 1), :]
        s_ref[0, pl.ds(d, 1), :] += m
        sq_ref[0, pl.ds(d, 1), :] += m * m
        mx_ref[0, pl.ds(d, 1), :] = jnp.maximum(mx_ref[0, pl.ds(d, 1), :], m)
        mn_ref[0, pl.ds(d, 1), :] = jnp.minimum(mn_ref[0, pl.ds(d, 1), :], m)
        if dg_ref is not None:
            dg_ref[0, pl.ds(d, 1), :] += 1.0
        return carry

    jax.lax.fori_loop(0, eblk, body, 0)


def _edge_aggregate(src4, dst4, A, B, with_deg):
    n, d = A.shape
    p, nblk, _, eblk = src4.shape
    outp = jax.ShapeDtypeStruct((p, n, d), jnp.float32)
    full = pl.BlockSpec((n, d), lambda i, j: (0, 0))
    acc = pl.BlockSpec((1, n, d), lambda i, j: (i, 0, 0))
    idx_spec = pl.BlockSpec((1, 1, 1, eblk), lambda i, j: (i, j, 0, 0),
                            memory_space=pltpu.SMEM)
    nout = 5 if with_deg else 4
    return pl.pallas_call(
        functools.partial(_edge_body, eblk=eblk, with_deg=with_deg),
        grid=(p, nblk),
        in_specs=[idx_spec, idx_spec, full, full],
        out_specs=[acc] * nout,
        out_shape=[outp] * nout,
        compiler_params=pltpu.CompilerParams(
            dimension_semantics=("parallel", "arbitrary")),
    )(src4, dst4, A, B)


def _k1_body(x_ref, s_ref, mx_ref, mn_ref, sq_ref, dg_ref,
             wu_ref, bu_ref, h_ref, ps_ref, pq_ref, *, n_nodes):
    deg = jnp.sum(dg_ref[:, :, 0:1], axis=0)
    degc = jnp.maximum(deg, 1.0)
    sm = jnp.sum(s_ref[...], axis=0)
    mean = sm / degc
    mx = jnp.max(mx_ref[...], axis=0)
    mx = jnp.where(jnp.isfinite(mx), mx, 0.0)
    mn = jnp.min(mn_ref[...], axis=0)
    mn = jnp.where(jnp.isfinite(mn), mn, 0.0)
    mean_sq = jnp.sum(sq_ref[...], axis=0) / degc
    var = jnp.maximum(mean_sq - mean * mean, 0.0)
    std = jnp.sqrt(var + 1e-30)

    logd = jnp.log(deg + 1.0)
    amp = logd / DELTA
    att = jnp.where(logd > 0, DELTA / jnp.maximum(logd, 1e-12), 0.0)

    x = x_ref[...]
    dim = x.shape[1]

    def blockw(k):
        return wu_ref[pl.ds(k * dim, dim), :]

    h = jnp.dot(x, blockw(0), preferred_element_type=jnp.float32)
    aggs = (mean, mx, mn, std, var, sm)
    for a, g in enumerate(aggs):
        h += jnp.dot(g, blockw(1 + a), preferred_element_type=jnp.float32)
        h += jnp.dot(g * amp, blockw(7 + a), preferred_element_type=jnp.float32)
        h += jnp.dot(g * att, blockw(13 + a), preferred_element_type=jnp.float32)
    h = (h + bu_ref[...]) * np.float32(np.sqrt(1.0 / n_nodes))
    h_ref[...] = h

    @pl.when(pl.program_id(0) == 0)
    def _init():
        ps_ref[...] = jnp.zeros_like(ps_ref)
        pq_ref[...] = jnp.zeros_like(pq_ref)

    ps_ref[...] += jnp.sum(h, axis=0, keepdims=True)
    pq_ref[...] += jnp.sum(h * h, axis=0, keepdims=True)


def _k2_body(h_ref, x_ref, mu_ref, mq_ref, gt_ref, bt_ref,
             wm_ref, bm_ref, y_ref, ps_ref, pq_ref, *, n_nodes):
    mu = mu_ref[...] / n_nodes
    var = mq_ref[...] / n_nodes - mu * mu
    h = h_ref[...]
    h = gt_ref[...] * (h - mu) * jax.lax.rsqrt(var + BN_EPS) + bt_ref[...]
    h = jnp.dot(h, wm_ref[...], preferred_element_type=jnp.float32) + bm_ref[...]
    h = jnp.where(h >= 0, h, 0.01 * h)
    y = h + x_ref[...]
    y_ref[...] = y

    @pl.when(pl.program_id(0) == 0)
    def _init():
        ps_ref[...] = jnp.zeros_like(ps_ref)
        pq_ref[...] = jnp.zeros_like(pq_ref)

    ps_ref[...] += jnp.sum(y, axis=0, keepdims=True)
    pq_ref[...] += jnp.sum(y * y, axis=0, keepdims=True)


def _k3_body(y_ref, x_ref, mu_ref, mq_ref, go_ref, bo_ref,
             w1_ref, w2_ref, bmsg_ref, h_ref, a_ref, b_ref, *, n_nodes):
    mu = mu_ref[...] / n_nodes
    var = mq_ref[...] / n_nodes - mu * mu
    y = y_ref[...]
    z = go_ref[...] * (y - mu) * jax.lax.rsqrt(var + BN_EPS) + bo_ref[...]
    h = jnp.maximum(z, 0.0) + x_ref[...]
    h_ref[...] = h
    a_ref[...] = jnp.dot(h, w1_ref[...], preferred_element_type=jnp.float32)
    b_ref[...] = (jnp.dot(h, w2_ref[...], preferred_element_type=jnp.float32)
                  + bmsg_ref[...])


def _ab_body(x_ref, w1_ref, w2_ref, bmsg_ref, a_ref, b_ref):
    x = x_ref[...]
    a_ref[...] = jnp.dot(x, w1_ref[...], preferred_element_type=jnp.float32)
    b_ref[...] = (jnp.dot(x, w2_ref[...], preferred_element_type=jnp.float32)
                  + bmsg_ref[...])


def kernel(x, edge_index, edge_attr, W_M, b_M, W_U, b_U, W_mix, b_mix,
           gamma_t, beta_t, gamma_o, beta_o):
    n, d = x.shape
    e = edge_index.shape[1]

    w1 = W_M[:, :d].T
    w2 = W_M[:, d:].T
    bm = b_M[None, :]
    wu = W_U.T
    bu = b_U[None, :]
    wmix = W_mix.T
    bmix = b_mix[None, :]
    gt = gamma_t[None, :]
    bt = beta_t[None, :]
    go = gamma_o[None, :]
    bo = beta_o[None, :]

    if e % 8000 == 0:
        p, eblk = 2, 4000
    else:
        p, eblk = 1, e
    nblk = e // (p * eblk)
    src4 = edge_index[0].reshape(p, nblk, 1, eblk)
    dst4 = edge_index[1].reshape(p, nblk, 1, eblk)

    rblk = 2000 if n % 2000 == 0 else n
    rgrid = n // rblk
    rb = pl.BlockSpec((rblk, d), lambda i: (i, 0))
    accb = pl.BlockSpec((p, rblk, d), lambda i: (0, i, 0))
    one = pl.BlockSpec((1, d), lambda i: (0, 0))
    wu_spec = pl.BlockSpec(wu.shape, lambda i: (0, 0))
    sq_spec = pl.BlockSpec((d, d), lambda i: (0, 0))
    f32 = jnp.float32

    k1 = pl.pallas_call(
        functools.partial(_k1_body, n_nodes=n),
        grid=(rgrid,),
        in_specs=[rb, accb, accb, accb, accb, accb, wu_spec, one],
        out_specs=[rb, one, one],
        out_shape=[jax.ShapeDtypeStruct((n, d), f32),
                   jax.ShapeDtypeStruct((1, d), f32),
                   jax.ShapeDtypeStruct((1, d), f32)],
    )

    k2 = pl.pallas_call(
        functools.partial(_k2_body, n_nodes=n),
        grid=(rgrid,),
        in_specs=[rb, rb, one, one, one, one, sq_spec, one],
        out_specs=[rb, one, one],
        out_shape=[jax.ShapeDtypeStruct((n, d), f32),
                   jax.ShapeDtypeStruct((1, d), f32),
                   jax.ShapeDtypeStruct((1, d), f32)],
    )

    k3 = pl.pallas_call(
        functools.partial(_k3_body, n_nodes=n),
        grid=(rgrid,),
        in_specs=[rb, rb, one, one, one, one, sq_spec, sq_spec, one],
        out_specs=[rb, rb, rb],
        out_shape=[jax.ShapeDtypeStruct((n, d), f32),
                   jax.ShapeDtypeStruct((n, d), f32),
                   jax.ShapeDtypeStruct((n, d), f32)],
    )

    ab0 = pl.pallas_call(
        _ab_body,
        grid=(rgrid,),
        in_specs=[rb, sq_spec, sq_spec, one],
        out_specs=[rb, rb],
        out_shape=[jax.ShapeDtypeStruct((n, d), f32),
                   jax.ShapeDtypeStruct((n, d), f32)],
    )

    h = x
    A, B = ab0(x, w1, w2, bm)
    dg = None
    for layer in range(NUM_LAYERS):
        if layer == 0:
            s, mx, mn, sq, dg = _edge_aggregate(src4, dst4, A, B, True)
        else:
            s, mx, mn, sq = _edge_aggregate(src4, dst4, A, B, False)
        hpre, ps, pq = k1(h, s, mx, mn, sq, dg, wu, bu)
        y, ys, yq = k2(hpre, h, ps, pq, gt, bt, wmix, bmix)
        h, A, B = k3(y, h, ys, yq, go, bo, w1, w2, bm)
    return h


# speedup vs baseline: 2.4726x; 1.6077x over previous
import functools

import jax
import jax.numpy as jnp
import numpy as np
from jax.experimental import pallas as pl
from jax.experimental.pallas import tpu as pltpu

DELTA = 2.5
NUM_LAYERS = 4
BN_EPS = 1e-5


def _edge_body(*refs, eblk, with_deg):
    if with_deg:
        (src_ref, dst_ref, a_ref, b_ref,
         s_ref, mx_ref, mn_ref, sq_ref, dg_ref) = refs
    else:
        src_ref, dst_ref, a_ref, b_ref, s_ref, mx_ref, mn_ref, sq_ref = refs
        dg_ref = None

    @pl.when(pl.program_id(1) == 0)
    def _init():
        s_ref[...] = jnp.zeros_like(s_ref)
        sq_ref[...] = jnp.zeros_like(sq_ref)
        mx_ref[...] = jnp.full_like(mx_ref, -jnp.inf)
        mn_ref[...] = jnp.full_like(mn_ref, jnp.inf)
        if dg_ref is not None:
            dg_ref[...] = jnp.zeros_like(dg_ref)

    def body(i, carry):
        s = src_ref[0, 0, 0, i]
        d = dst_ref[0, 0, 0, i]
        m = a_ref[pl.ds(s, 1), :] + b_ref[pl.ds(d, 1), :]
        s_ref[0, pl.ds(d, 1), :] += m
        sq_ref[0, pl.ds(d, 1), :] += m * m
        mx_ref[0, pl.ds(d, 1), :] = jnp.maximum(mx_ref[0, pl.ds(d, 1), :], m)
        mn_ref[0, pl.ds(d, 1), :] = jnp.minimum(mn_ref[0, pl.ds(d, 1), :], m)
        if dg_ref is not None:
            dg_ref[0, pl.ds(d, 1), :] += 1.0
        return carry

    jax.lax.fori_loop(0, eblk, body, 0, unroll=8)


def _edge_aggregate(src4, dst4, A, B, with_deg):
    n, d = A.shape
    p, nblk, _, eblk = src4.shape
    outp = jax.ShapeDtypeStruct((p, n, d), jnp.float32)
    full = pl.BlockSpec((n, d), lambda i, j: (0, 0))
    acc = pl.BlockSpec((1, n, d), lambda i, j: (i, 0, 0))
    idx_spec = pl.BlockSpec((1, 1, 1, eblk), lambda i, j: (i, j, 0, 0),
                            memory_space=pltpu.SMEM)
    nout = 5 if with_deg else 4
    return pl.pallas_call(
        functools.partial(_edge_body, eblk=eblk, with_deg=with_deg),
        grid=(p, nblk),
        in_specs=[idx_spec, idx_spec, full, full],
        out_specs=[acc] * nout,
        out_shape=[outp] * nout,
        compiler_params=pltpu.CompilerParams(
            dimension_semantics=("parallel", "arbitrary")),
    )(src4, dst4, A, B)


def _k1_body(x_ref, s_ref, mx_ref, mn_ref, sq_ref, dg_ref,
             wu_ref, bu_ref, h_ref, ps_ref, pq_ref, *, n_nodes):
    deg = jnp.sum(dg_ref[:, :, 0:1], axis=0)
    degc = jnp.maximum(deg, 1.0)
    sm = jnp.sum(s_ref[...], axis=0)
    mean = sm / degc
    mx = jnp.max(mx_ref[...], axis=0)
    mx = jnp.where(jnp.isfinite(mx), mx, 0.0)
    mn = jnp.min(mn_ref[...], axis=0)
    mn = jnp.where(jnp.isfinite(mn), mn, 0.0)
    mean_sq = jnp.sum(sq_ref[...], axis=0) / degc
    var = jnp.maximum(mean_sq - mean * mean, 0.0)
    std = jnp.sqrt(var + 1e-30)

    logd = jnp.log(deg + 1.0)
    amp = logd / DELTA
    att = jnp.where(logd > 0, DELTA / jnp.maximum(logd, 1e-12), 0.0)

    x = x_ref[...]
    dim = x.shape[1]

    def blockw(k):
        return wu_ref[pl.ds(k * dim, dim), :]

    h = jnp.dot(x, blockw(0), preferred_element_type=jnp.float32)
    aggs = (mean, mx, mn, std, var, sm)
    for a, g in enumerate(aggs):
        h += jnp.dot(g, blockw(1 + a), preferred_element_type=jnp.float32)
        h += jnp.dot(g * amp, blockw(7 + a), preferred_element_type=jnp.float32)
        h += jnp.dot(g * att, blockw(13 + a), preferred_element_type=jnp.float32)
    h = (h + bu_ref[...]) * np.float32(np.sqrt(1.0 / n_nodes))
    h_ref[...] = h

    @pl.when(pl.program_id(0) == 0)
    def _init():
        ps_ref[...] = jnp.zeros_like(ps_ref)
        pq_ref[...] = jnp.zeros_like(pq_ref)

    ps_ref[...] += jnp.sum(h, axis=0, keepdims=True)
    pq_ref[...] += jnp.sum(h * h, axis=0, keepdims=True)


def _k2_body(h_ref, x_ref, mu_ref, mq_ref, gt_ref, bt_ref,
             wm_ref, bm_ref, y_ref, ps_ref, pq_ref, *, n_nodes):
    mu = mu_ref[...] / n_nodes
    var = mq_ref[...] / n_nodes - mu * mu
    h = h_ref[...]
    h = gt_ref[...] * (h - mu) * jax.lax.rsqrt(var + BN_EPS) + bt_ref[...]
    h = jnp.dot(h, wm_ref[...], preferred_element_type=jnp.float32) + bm_ref[...]
    h = jnp.where(h >= 0, h, 0.01 * h)
    y = h + x_ref[...]
    y_ref[...] = y

    @pl.when(pl.program_id(0) == 0)
    def _init():
        ps_ref[...] = jnp.zeros_like(ps_ref)
        pq_ref[...] = jnp.zeros_like(pq_ref)

    ps_ref[...] += jnp.sum(y, axis=0, keepdims=True)
    pq_ref[...] += jnp.sum(y * y, axis=0, keepdims=True)


def _k3_body(y_ref, x_ref, mu_ref, mq_ref, go_ref, bo_ref,
             w1_ref, w2_ref, bmsg_ref, h_ref, a_ref, b_ref, *, n_nodes):
    mu = mu_ref[...] / n_nodes
    var = mq_ref[...] / n_nodes - mu * mu
    y = y_ref[...]
    z = go_ref[...] * (y - mu) * jax.lax.rsqrt(var + BN_EPS) + bo_ref[...]
    h = jnp.maximum(z, 0.0) + x_ref[...]
    h_ref[...] = h
    a_ref[...] = jnp.dot(h, w1_ref[...], preferred_element_type=jnp.float32)
    b_ref[...] = (jnp.dot(h, w2_ref[...], preferred_element_type=jnp.float32)
                  + bmsg_ref[...])


def _ab_body(x_ref, w1_ref, w2_ref, bmsg_ref, a_ref, b_ref):
    x = x_ref[...]
    a_ref[...] = jnp.dot(x, w1_ref[...], preferred_element_type=jnp.float32)
    b_ref[...] = (jnp.dot(x, w2_ref[...], preferred_element_type=jnp.float32)
                  + bmsg_ref[...])


def kernel(x, edge_index, edge_attr, W_M, b_M, W_U, b_U, W_mix, b_mix,
           gamma_t, beta_t, gamma_o, beta_o):
    n, d = x.shape
    e = edge_index.shape[1]

    w1 = W_M[:, :d].T
    w2 = W_M[:, d:].T
    bm = b_M[None, :]
    wu = W_U.T
    bu = b_U[None, :]
    wmix = W_mix.T
    bmix = b_mix[None, :]
    gt = gamma_t[None, :]
    bt = beta_t[None, :]
    go = gamma_o[None, :]
    bo = beta_o[None, :]

    if e % 8000 == 0:
        p, eblk = 2, 4000
    else:
        p, eblk = 1, e
    nblk = e // (p * eblk)
    src4 = edge_index[0].reshape(p, nblk, 1, eblk)
    dst4 = edge_index[1].reshape(p, nblk, 1, eblk)

    rblk = 2000 if n % 2000 == 0 else n
    rgrid = n // rblk
    rb = pl.BlockSpec((rblk, d), lambda i: (i, 0))
    accb = pl.BlockSpec((p, rblk, d), lambda i: (0, i, 0))
    one = pl.BlockSpec((1, d), lambda i: (0, 0))
    wu_spec = pl.BlockSpec(wu.shape, lambda i: (0, 0))
    sq_spec = pl.BlockSpec((d, d), lambda i: (0, 0))
    f32 = jnp.float32

    k1 = pl.pallas_call(
        functools.partial(_k1_body, n_nodes=n),
        grid=(rgrid,),
        in_specs=[rb, accb, accb, accb, accb, accb, wu_spec, one],
        out_specs=[rb, one, one],
        out_shape=[jax.ShapeDtypeStruct((n, d), f32),
                   jax.ShapeDtypeStruct((1, d), f32),
                   jax.ShapeDtypeStruct((1, d), f32)],
    )

    k2 = pl.pallas_call(
        functools.partial(_k2_body, n_nodes=n),
        grid=(rgrid,),
        in_specs=[rb, rb, one, one, one, one, sq_spec, one],
        out_specs=[rb, one, one],
        out_shape=[jax.ShapeDtypeStruct((n, d), f32),
                   jax.ShapeDtypeStruct((1, d), f32),
                   jax.ShapeDtypeStruct((1, d), f32)],
    )

    k3 = pl.pallas_call(
        functools.partial(_k3_body, n_nodes=n),
        grid=(rgrid,),
        in_specs=[rb, rb, one, one, one, one, sq_spec, sq_spec, one],
        out_specs=[rb, rb, rb],
        out_shape=[jax.ShapeDtypeStruct((n, d), f32),
                   jax.ShapeDtypeStruct((n, d), f32),
                   jax.ShapeDtypeStruct((n, d), f32)],
    )

    ab0 = pl.pallas_call(
        _ab_body,
        grid=(rgrid,),
        in_specs=[rb, sq_spec, sq_spec, one],
        out_specs=[rb, rb],
        out_shape=[jax.ShapeDtypeStruct((n, d), f32),
                   jax.ShapeDtypeStruct((n, d), f32)],
    )

    h = x
    A, B = ab0(x, w1, w2, bm)
    dg = None
    for layer in range(NUM_LAYERS):
        if layer == 0:
            s, mx, mn, sq, dg = _edge_aggregate(src4, dst4, A, B, True)
        else:
            s, mx, mn, sq = _edge_aggregate(src4, dst4, A, B, False)
        hpre, ps, pq = k1(h, s, mx, mn, sq, dg, wu, bu)
        y, ys, yq = k2(hpre, h, ps, pq, gt, bt, wmix, bmix)
        h, A, B = k3(y, h, ys, yq, go, bo, w1, w2, bm)
    return h


# speedup vs baseline: 2.5412x; 1.0277x over previous
import functools

import jax
import jax.numpy as jnp
import numpy as np
from jax.experimental import pallas as pl
from jax.experimental.pallas import tpu as pltpu

DELTA = 2.5
NUM_LAYERS = 4
BN_EPS = 1e-5


def _edge_body(*refs, eblk, with_deg):
    if with_deg:
        (src_ref, dst_ref, a_ref, b_ref,
         s_ref, mx_ref, mn_ref, sq_ref, dg_ref) = refs
    else:
        src_ref, dst_ref, a_ref, b_ref, s_ref, mx_ref, mn_ref, sq_ref = refs
        dg_ref = None

    @pl.when(pl.program_id(1) == 0)
    def _init():
        s_ref[...] = jnp.zeros_like(s_ref)
        sq_ref[...] = jnp.zeros_like(sq_ref)
        mx_ref[...] = jnp.full_like(mx_ref, -jnp.inf)
        mn_ref[...] = jnp.full_like(mn_ref, jnp.inf)
        if dg_ref is not None:
            dg_ref[...] = jnp.zeros_like(dg_ref)

    def body(i, carry):
        s = src_ref[0, 0, 0, i]
        d = dst_ref[0, 0, 0, i]
        m = a_ref[pl.ds(s, 1), :] + b_ref[pl.ds(d, 1), :]
        s_ref[0, pl.ds(d, 1), :] += m
        sq_ref[0, pl.ds(d, 1), :] += m * m
        mx_ref[0, pl.ds(d, 1), :] = jnp.maximum(mx_ref[0, pl.ds(d, 1), :], m)
        mn_ref[0, pl.ds(d, 1), :] = jnp.minimum(mn_ref[0, pl.ds(d, 1), :], m)
        if dg_ref is not None:
            dg_ref[0, pl.ds(d, 1), :] += 1.0
        return carry

    jax.lax.fori_loop(0, eblk, body, 0, unroll=16)


def _edge_aggregate(src4, dst4, A, B, with_deg):
    n, d = A.shape
    p, nblk, _, eblk = src4.shape
    outp = jax.ShapeDtypeStruct((p, n, d), jnp.float32)
    full = pl.BlockSpec((n, d), lambda i, j: (0, 0))
    acc = pl.BlockSpec((1, n, d), lambda i, j: (i, 0, 0))
    idx_spec = pl.BlockSpec((1, 1, 1, eblk), lambda i, j: (i, j, 0, 0),
                            memory_space=pltpu.SMEM)
    nout = 5 if with_deg else 4
    return pl.pallas_call(
        functools.partial(_edge_body, eblk=eblk, with_deg=with_deg),
        grid=(p, nblk),
        in_specs=[idx_spec, idx_spec, full, full],
        out_specs=[acc] * nout,
        out_shape=[outp] * nout,
        compiler_params=pltpu.CompilerParams(
            dimension_semantics=("parallel", "arbitrary")),
    )(src4, dst4, A, B)


def _k1_body(x_ref, s_ref, mx_ref, mn_ref, sq_ref, dg_ref,
             wu_ref, bu_ref, h_ref, ps_ref, pq_ref, *, n_nodes):
    deg = jnp.sum(dg_ref[:, :, 0:1], axis=0)
    degc = jnp.maximum(deg, 1.0)
    sm = jnp.sum(s_ref[...], axis=0)
    mean = sm / degc
    mx = jnp.max(mx_ref[...], axis=0)
    mx = jnp.where(jnp.isfinite(mx), mx, 0.0)
    mn = jnp.min(mn_ref[...], axis=0)
    mn = jnp.where(jnp.isfinite(mn), mn, 0.0)
    mean_sq = jnp.sum(sq_ref[...], axis=0) / degc
    var = jnp.maximum(mean_sq - mean * mean, 0.0)
    std = jnp.sqrt(var + 1e-30)

    logd = jnp.log(deg + 1.0)
    amp = logd / DELTA
    att = jnp.where(logd > 0, DELTA / jnp.maximum(logd, 1e-12), 0.0)

    x = x_ref[...]
    dim = x.shape[1]

    def blockw(k):
        return wu_ref[pl.ds(k * dim, dim), :]

    h = jnp.dot(x, blockw(0), preferred_element_type=jnp.float32)
    aggs = (mean, mx, mn, std, var, sm)
    for a, g in enumerate(aggs):
        h += jnp.dot(g, blockw(1 + a), preferred_element_type=jnp.float32)
        h += jnp.dot(g * amp, blockw(7 + a), preferred_element_type=jnp.float32)
        h += jnp.dot(g * att, blockw(13 + a), preferred_element_type=jnp.float32)
    h = (h + bu_ref[...]) * np.float32(np.sqrt(1.0 / n_nodes))
    h_ref[...] = h

    @pl.when(pl.program_id(0) == 0)
    def _init():
        ps_ref[...] = jnp.zeros_like(ps_ref)
        pq_ref[...] = jnp.zeros_like(pq_ref)

    ps_ref[...] += jnp.sum(h, axis=0, keepdims=True)
    pq_ref[...] += jnp.sum(h * h, axis=0, keepdims=True)


def _k2_body(h_ref, x_ref, mu_ref, mq_ref, gt_ref, bt_ref,
             wm_ref, bm_ref, y_ref, ps_ref, pq_ref, *, n_nodes):
    mu = mu_ref[...] / n_nodes
    var = mq_ref[...] / n_nodes - mu * mu
    h = h_ref[...]
    h = gt_ref[...] * (h - mu) * jax.lax.rsqrt(var + BN_EPS) + bt_ref[...]
    h = jnp.dot(h, wm_ref[...], preferred_element_type=jnp.float32) + bm_ref[...]
    h = jnp.where(h >= 0, h, 0.01 * h)
    y = h + x_ref[...]
    y_ref[...] = y

    @pl.when(pl.program_id(0) == 0)
    def _init():
        ps_ref[...] = jnp.zeros_like(ps_ref)
        pq_ref[...] = jnp.zeros_like(pq_ref)

    ps_ref[...] += jnp.sum(y, axis=0, keepdims=True)
    pq_ref[...] += jnp.sum(y * y, axis=0, keepdims=True)


def _k3_body(y_ref, x_ref, mu_ref, mq_ref, go_ref, bo_ref,
             w1_ref, w2_ref, bmsg_ref, h_ref, a_ref, b_ref, *, n_nodes):
    mu = mu_ref[...] / n_nodes
    var = mq_ref[...] / n_nodes - mu * mu
    y = y_ref[...]
    z = go_ref[...] * (y - mu) * jax.lax.rsqrt(var + BN_EPS) + bo_ref[...]
    h = jnp.maximum(z, 0.0) + x_ref[...]
    h_ref[...] = h
    a_ref[...] = jnp.dot(h, w1_ref[...], preferred_element_type=jnp.float32)
    b_ref[...] = (jnp.dot(h, w2_ref[...], preferred_element_type=jnp.float32)
                  + bmsg_ref[...])


def _ab_body(x_ref, w1_ref, w2_ref, bmsg_ref, a_ref, b_ref):
    x = x_ref[...]
    a_ref[...] = jnp.dot(x, w1_ref[...], preferred_element_type=jnp.float32)
    b_ref[...] = (jnp.dot(x, w2_ref[...], preferred_element_type=jnp.float32)
                  + bmsg_ref[...])


def kernel(x, edge_index, edge_attr, W_M, b_M, W_U, b_U, W_mix, b_mix,
           gamma_t, beta_t, gamma_o, beta_o):
    n, d = x.shape
    e = edge_index.shape[1]

    w1 = W_M[:, :d].T
    w2 = W_M[:, d:].T
    bm = b_M[None, :]
    wu = W_U.T
    bu = b_U[None, :]
    wmix = W_mix.T
    bmix = b_mix[None, :]
    gt = gamma_t[None, :]
    bt = beta_t[None, :]
    go = gamma_o[None, :]
    bo = beta_o[None, :]

    if e % 8000 == 0:
        p, eblk = 2, 4000
    else:
        p, eblk = 1, e
    nblk = e // (p * eblk)
    src4 = edge_index[0].reshape(p, nblk, 1, eblk)
    dst4 = edge_index[1].reshape(p, nblk, 1, eblk)

    rblk = 2000 if n % 2000 == 0 else n
    rgrid = n // rblk
    rb = pl.BlockSpec((rblk, d), lambda i: (i, 0))
    accb = pl.BlockSpec((p, rblk, d), lambda i: (0, i, 0))
    one = pl.BlockSpec((1, d), lambda i: (0, 0))
    wu_spec = pl.BlockSpec(wu.shape, lambda i: (0, 0))
    sq_spec = pl.BlockSpec((d, d), lambda i: (0, 0))
    f32 = jnp.float32

    k1 = pl.pallas_call(
        functools.partial(_k1_body, n_nodes=n),
        grid=(rgrid,),
        in_specs=[rb, accb, accb, accb, accb, accb, wu_spec, one],
        out_specs=[rb, one, one],
        out_shape=[jax.ShapeDtypeStruct((n, d), f32),
                   jax.ShapeDtypeStruct((1, d), f32),
                   jax.ShapeDtypeStruct((1, d), f32)],
    )

    k2 = pl.pallas_call(
        functools.partial(_k2_body, n_nodes=n),
        grid=(rgrid,),
        in_specs=[rb, rb, one, one, one, one, sq_spec, one],
        out_specs=[rb, one, one],
        out_shape=[jax.ShapeDtypeStruct((n, d), f32),
                   jax.ShapeDtypeStruct((1, d), f32),
                   jax.ShapeDtypeStruct((1, d), f32)],
    )

    k3 = pl.pallas_call(
        functools.partial(_k3_body, n_nodes=n),
        grid=(rgrid,),
        in_specs=[rb, rb, one, one, one, one, sq_spec, sq_spec, one],
        out_specs=[rb, rb, rb],
        out_shape=[jax.ShapeDtypeStruct((n, d), f32),
                   jax.ShapeDtypeStruct((n, d), f32),
                   jax.ShapeDtypeStruct((n, d), f32)],
    )

    ab0 = pl.pallas_call(
        _ab_body,
        grid=(rgrid,),
        in_specs=[rb, sq_spec, sq_spec, one],
        out_specs=[rb, rb],
        out_shape=[jax.ShapeDtypeStruct((n, d), f32),
                   jax.ShapeDtypeStruct((n, d), f32)],
    )

    h = x
    A, B = ab0(x, w1, w2, bm)
    dg = None
    for layer in range(NUM_LAYERS):
        if layer == 0:
            s, mx, mn, sq, dg = _edge_aggregate(src4, dst4, A, B, True)
        else:
            s, mx, mn, sq = _edge_aggregate(src4, dst4, A, B, False)
        hpre, ps, pq = k1(h, s, mx, mn, sq, dg, wu, bu)
        y, ys, yq = k2(hpre, h, ps, pq, gt, bt, wmix, bmix)
        h, A, B = k3(y, h, ys, yq, go, bo, w1, w2, bm)
    return h
